# Initial kernel scaffold; baseline (speedup 1.0000x reference)
import jax
import jax.numpy as jnp
from jax.experimental import pallas as pl


def kernel(inputs):
    raise NotImplementedError("write your pallas kernel here")


# speedup vs baseline: 84.0929x; 84.0929x over previous
import functools

import jax
import jax.numpy as jnp
from jax import lax
from jax.experimental import pallas as pl
from jax.experimental.pallas import tpu as pltpu
from jax.experimental.pallas import tpu_sc as plsc

B, T, V = 16, 256, 96
W = 16
BLANK = V - 1
NV = V // 16


def _ls_body(x_ref, o_ref):
    x = x_ref[...]
    m = jnp.max(x, axis=-1, keepdims=True)
    s = x - m
    o_ref[...] = s - jnp.log(jnp.sum(jnp.exp(s), axis=-1, keepdims=True))


def _log_softmax_tc(x):
    return pl.pallas_call(
        _ls_body,
        out_shape=jax.ShapeDtypeStruct((B, T, V), jnp.float32),
    )(x)


def _take16(v, idx):
    return jnp.take_along_axis(v, idx, axis=0, mode="promise_in_bounds")


def _merge(a, b):
    av, ap = a
    bv, bp = b
    bvr = jnp.flip(bv, axis=0)
    bpr = jnp.flip(bp, axis=0)
    cm = av >= bvr
    mv = jnp.where(cm, av, bvr)
    mp = jnp.where(cm, ap, bpr)
    sv, sp = plsc.sort_key_val(mv, mp, descending=True)
    return sv, sp


def _sc_body(logp_hbm, dec_hbm, prob_hbm, logp_v, path_v, rec_v, dec_v, prob_v):
    cid = lax.axis_index("c")
    sid = lax.axis_index("s")
    wid = sid * 2 + cid

    @pl.when(wid < B)
    def _():
        b = wid
        pltpu.sync_copy(logp_hbm.at[pl.ds(b * T * V, T * V)], logp_v)
        iota = lax.iota(jnp.int32, 16)

        def top16(t):
            parts = []
            for i in range(NV):
                v = logp_v[pl.ds(t * V + 16 * i, 16)]
                p = iota + 16 * i
                parts.append(plsc.sort_key_val(v, p, descending=True))
            m01 = _merge(parts[0], parts[1])
            m23 = _merge(parts[2], parts[3])
            m45 = _merge(parts[4], parts[5])
            return _merge(_merge(m01, m23), m45)

        lptv0, lpti0 = top16(0)
        rec_v[pl.ds(0, 16)] = lpti0

        def step(t, scores):
            lptv, lpti = top16(t)
            rows = []
            for w in range(W):
                sw = _take16(scores, jnp.full((16,), w, jnp.int32))
                rows.append((sw + lptv, iota + 16 * w))
            while len(rows) > 1:
                rows = [_merge(rows[i], rows[i + 1])
                        for i in range(0, len(rows), 2)]
            rv, rp = rows[0]
            wpar = rp >> 4
            j = rp & 15
            tok = _take16(lpti, j)
            rec_v[pl.ds(t * 16, 16)] = (wpar << 7) | tok
            return rv

        scores = lax.fori_loop(1, T, step, lptv0)

        lane0 = iota == 0

        def bstep(k, wv):
            t = T - 1 - k
            r = plsc.load_gather(rec_v, [jnp.full((16,), t * 16, jnp.int32) + wv])
            plsc.store_scatter(path_v, [jnp.full((16,), t, jnp.int32)],
                               r & 127, mask=lane0)
            return r >> 7

        wv = lax.fori_loop(0, T - 1, bstep, jnp.zeros((16,), jnp.int32))
        r0 = plsc.load_gather(rec_v, [wv])
        plsc.store_scatter(path_v, [jnp.zeros((16,), jnp.int32)],
                           r0 & 127, mask=lane0)

        for i in range(T // 16):
            dec_v[pl.ds(16 * i, 16)] = jnp.full((16,), -1, jnp.int32)
        running = jnp.int32(0)
        for i in range(T // 16):
            cur = path_v[pl.ds(16 * i, 16)]
            if i == 0:
                prev = plsc.load_gather(path_v, [jnp.maximum(iota - 1, 0)])
                prev = jnp.where(lane0, -1, prev)
            else:
                prev = plsc.load_gather(path_v, [iota + (16 * i - 1)])
            keep = (cur != prev) & (cur != BLANK)
            kint = jnp.where(keep, 1, 0).astype(jnp.int32)
            pos = plsc.cumsum(kint) + running - 1
            plsc.store_scatter(dec_v, [pos], cur, mask=keep)
            running = running + jnp.sum(kint)

        prob_v[...] = jnp.exp(scores)
        pltpu.sync_copy(dec_v, dec_hbm.at[pl.ds(b * T, T)])
        pltpu.sync_copy(prob_v, prob_hbm.at[pl.ds(b * 16, 16)])


@functools.cache
def _sc_decode():
    return pl.kernel(
        _sc_body,
        out_type=[
            jax.ShapeDtypeStruct((B * T,), jnp.int32),
            jax.ShapeDtypeStruct((B * 16,), jnp.float32),
        ],
        mesh=plsc.VectorSubcoreMesh(core_axis_name="c", subcore_axis_name="s"),
        compiler_params=pltpu.CompilerParams(needs_layout_passes=False),
        scratch_types=[
            pltpu.VMEM((T * V,), jnp.float32),
            pltpu.VMEM((T,), jnp.int32),
            pltpu.VMEM((T * 16,), jnp.int32),
            pltpu.VMEM((T,), jnp.int32),
            pltpu.VMEM((16,), jnp.float32),
        ],
    )


def kernel(inputs):
    logp = _log_softmax_tc(inputs)
    dec, prob = _sc_decode()(logp.reshape(B * T * V))
    decoded = dec.reshape(B, 1, T)
    probability = prob.reshape(B, 16)[:, :1]
    return decoded, probability
